# baseline (speedup 1.0000x reference)
import functools

import jax
import jax.numpy as jnp
from jax import lax
from jax.experimental import pallas as pl
from jax.experimental.pallas import tpu as pltpu
from jax.experimental.pallas import tpu_sc as plsc

EMB_DIM = 64
BATCH = 16384
CTX = 20
NW = 32
CB = 32
ROWS_PER_W = 2 * BATCH // NW
CHUNKS = ROWS_PER_W // CB


def _sc_partials(u_weight, v_weight, all_u, all_v):
    mesh = plsc.VectorSubcoreMesh(core_axis_name="c", subcore_axis_name="s")

    @functools.partial(
        pl.kernel,
        mesh=mesh,
        compiler_params=pltpu.CompilerParams(use_tc_tiling_on_sc=False),
        out_type=jax.ShapeDtypeStruct((2 * BATCH, 16), jnp.float32),
        scratch_types=[
            pltpu.VMEM((CB, CTX), jnp.int32),
            pltpu.VMEM((CB,), jnp.int32),
            pltpu.VMEM((CB, CTX, EMB_DIM), jnp.float32),
            pltpu.VMEM((CB, EMB_DIM), jnp.float32),
            pltpu.VMEM((CB, 16), jnp.float32),
            pltpu.SemaphoreType.DMA,
        ],
    )
    def k(u_hbm, v_hbm, uidx_hbm, vidx_hbm, out_hbm,
          uidx_v, vidx_v, rows_v, vrows_v, part_v, sem):
        wid = lax.axis_index("s") * 2 + lax.axis_index("c")
        base = wid * ROWS_PER_W

        def chunk_body(ci, _):
            r0 = base + ci * CB
            pltpu.sync_copy(uidx_hbm.at[pl.ds(r0, CB)], uidx_v)
            pltpu.sync_copy(vidx_hbm.at[pl.ds(r0, CB)], vidx_v)

            vcp = pltpu.async_copy(v_hbm.at[vidx_v], vrows_v, sem)

            def fire(b, carry):
                pltpu.async_copy(u_hbm.at[uidx_v.at[b]], rows_v.at[b], sem)
                return carry

            lax.fori_loop(0, CB, fire, 0)
            vcp.wait()

            def drain(b, carry):
                pltpu.make_async_copy(
                    u_hbm.at[uidx_v.at[b]], rows_v.at[b], sem).wait()
                return carry

            lax.fori_loop(0, CB, drain, 0)

            def row_body(b, carry):
                def ctx_body(c, accs):
                    a0, a1, a2, a3 = accs
                    a0 = a0 + rows_v[b, c, pl.ds(0, 16)]
                    a1 = a1 + rows_v[b, c, pl.ds(16, 16)]
                    a2 = a2 + rows_v[b, c, pl.ds(32, 16)]
                    a3 = a3 + rows_v[b, c, pl.ds(48, 16)]
                    return (a0, a1, a2, a3)

                z = jnp.zeros((16,), jnp.float32)
                a0, a1, a2, a3 = lax.fori_loop(0, CTX, ctx_body, (z, z, z, z))
                p = (a0 * vrows_v[b, pl.ds(0, 16)]
                     + a1 * vrows_v[b, pl.ds(16, 16)]
                     + a2 * vrows_v[b, pl.ds(32, 16)]
                     + a3 * vrows_v[b, pl.ds(48, 16)])
                part_v[b, :] = p
                return carry

            lax.fori_loop(0, CB, row_body, 0)
            pltpu.sync_copy(part_v, out_hbm.at[pl.ds(r0, CB)])
            return 0

        lax.fori_loop(0, CHUNKS, chunk_body, 0)

    return k(u_weight, v_weight, all_u, all_v)


def _tc_loss(partials):
    def body(p_ref, o_ref):
        x = p_ref[...]
        s = jnp.sum(x, axis=1, keepdims=True)
        row = lax.broadcasted_iota(jnp.int32, (2 * BATCH, 1), 0)
        z = jnp.where(row < BATCH, s, -s)
        l = jnp.minimum(z, 0.0) - jnp.log1p(jnp.exp(-jnp.abs(z)))
        o_ref[0, 0] = -jnp.sum(l)

    out = pl.pallas_call(
        body,
        out_shape=jax.ShapeDtypeStruct((1, 1), jnp.float32),
        out_specs=pl.BlockSpec(memory_space=pltpu.SMEM),
    )(partials)
    return out[0, 0]


def kernel(pos_u, pos_v, neg_u, neg_v, u_weight, v_weight):
    all_u = jnp.concatenate([pos_u, neg_u], axis=0)
    all_v = jnp.concatenate([pos_v, neg_v], axis=0)
    partials = _sc_partials(u_weight, v_weight, all_u, all_v)
    return _tc_loss(partials)


# speedup vs baseline: 1.0253x; 1.0253x over previous
import functools

import jax
import jax.numpy as jnp
from jax import lax
from jax.experimental import pallas as pl
from jax.experimental.pallas import tpu as pltpu
from jax.experimental.pallas import tpu_sc as plsc

EMB_DIM = 64
BATCH = 16384
CTX = 20
NW = 32
CB = 32
ROWS_PER_W = 2 * BATCH // NW
CHUNKS = ROWS_PER_W // CB
GPC = CB * CTX // 128
IDXROWS_PER_W = ROWS_PER_W * CTX // 128
IDXROWS_PER_CHUNK = CB * CTX // 128


def _sc_partials(u_weight, v_weight, all_u2, all_v):
    mesh = plsc.VectorSubcoreMesh(core_axis_name="c", subcore_axis_name="s")

    @functools.partial(
        pl.kernel,
        mesh=mesh,
        compiler_params=pltpu.CompilerParams(use_tc_tiling_on_sc=False),
        out_type=jax.ShapeDtypeStruct((2 * BATCH, 16), jnp.float32),
        scratch_types=[
            pltpu.VMEM((IDXROWS_PER_CHUNK, 128), jnp.int32),
            pltpu.VMEM((CB,), jnp.int32),
            pltpu.VMEM((CB * CTX, EMB_DIM), jnp.float32),
            pltpu.VMEM((CB, EMB_DIM), jnp.float32),
            pltpu.VMEM((CB, 16), jnp.float32),
            pltpu.SemaphoreType.DMA,
        ],
    )
    def k(u_hbm, v_hbm, uidx_hbm, vidx_hbm, out_hbm,
          uidx_v, vidx_v, rows_v, vrows_v, part_v, sem):
        wid = lax.axis_index("s") * 2 + lax.axis_index("c")
        base = wid * ROWS_PER_W
        ibase = wid * IDXROWS_PER_W

        def chunk_body(ci, _):
            r0 = base + ci * CB
            pltpu.sync_copy(
                uidx_hbm.at[pl.ds(ibase + ci * IDXROWS_PER_CHUNK,
                                  IDXROWS_PER_CHUNK)], uidx_v)
            pltpu.sync_copy(vidx_hbm.at[pl.ds(r0, CB)], vidx_v)

            vcp = pltpu.async_copy(v_hbm.at[vidx_v], vrows_v, sem)
            ucps = [
                pltpu.async_copy(
                    u_hbm.at[uidx_v.at[j]],
                    rows_v.at[pl.ds(j * 128, 128)], sem)
                for j in range(GPC)
            ]
            vcp.wait()
            for cp in ucps:
                cp.wait()

            def row_body(b, carry):
                r = b * CTX
                a0 = rows_v[r, pl.ds(0, 16)]
                a1 = rows_v[r, pl.ds(16, 16)]
                a2 = rows_v[r, pl.ds(32, 16)]
                a3 = rows_v[r, pl.ds(48, 16)]
                for c in range(1, CTX):
                    a0 = a0 + rows_v[r + c, pl.ds(0, 16)]
                    a1 = a1 + rows_v[r + c, pl.ds(16, 16)]
                    a2 = a2 + rows_v[r + c, pl.ds(32, 16)]
                    a3 = a3 + rows_v[r + c, pl.ds(48, 16)]
                p = (a0 * vrows_v[b, pl.ds(0, 16)]
                     + a1 * vrows_v[b, pl.ds(16, 16)]
                     + a2 * vrows_v[b, pl.ds(32, 16)]
                     + a3 * vrows_v[b, pl.ds(48, 16)])
                part_v[b, :] = p
                return carry

            lax.fori_loop(0, CB, row_body, 0)
            pltpu.sync_copy(part_v, out_hbm.at[pl.ds(r0, CB)])
            return 0

        lax.fori_loop(0, CHUNKS, chunk_body, 0)

    return k(u_weight, v_weight, all_u2, all_v)


def _tc_loss(partials):
    def body(p_ref, o_ref):
        x = p_ref[...]
        s = jnp.sum(x, axis=1, keepdims=True)
        row = lax.broadcasted_iota(jnp.int32, (2 * BATCH, 1), 0)
        z = jnp.where(row < BATCH, s, -s)
        l = jnp.minimum(z, 0.0) - jnp.log1p(jnp.exp(-jnp.abs(z)))
        o_ref[0, 0] = -jnp.sum(l)

    out = pl.pallas_call(
        body,
        out_shape=jax.ShapeDtypeStruct((1, 1), jnp.float32),
        out_specs=pl.BlockSpec(memory_space=pltpu.SMEM),
    )(partials)
    return out[0, 0]


def kernel(pos_u, pos_v, neg_u, neg_v, u_weight, v_weight):
    all_u2 = jnp.concatenate([pos_u, neg_u], axis=0).reshape(-1, 128)
    all_v = jnp.concatenate([pos_v, neg_v], axis=0)
    partials = _sc_partials(u_weight, v_weight, all_u2, all_v)
    return _tc_loss(partials)


# speedup vs baseline: 1.0722x; 1.0458x over previous
import functools

import jax
import jax.numpy as jnp
from jax import lax
from jax.experimental import pallas as pl
from jax.experimental.pallas import tpu as pltpu
from jax.experimental.pallas import tpu_sc as plsc

EMB_DIM = 64
BATCH = 16384
CTX = 20
NW = 32
CB = 32
ROWS_PER_W = 2 * BATCH // NW
CHUNKS = ROWS_PER_W // CB


def _sc_partials(u_weight, v_weight, all_u, all_v):
    mesh = plsc.VectorSubcoreMesh(core_axis_name="c", subcore_axis_name="s")

    @functools.partial(
        pl.kernel,
        mesh=mesh,
        compiler_params=pltpu.CompilerParams(use_tc_tiling_on_sc=False),
        out_type=jax.ShapeDtypeStruct((2 * BATCH, 16), jnp.float32),
        scratch_types=[
            pltpu.VMEM((2, CB, CTX), jnp.int32),
            pltpu.VMEM((2, CB), jnp.int32),
            pltpu.VMEM((2, CB, CTX, EMB_DIM), jnp.float32),
            pltpu.VMEM((2, CB, EMB_DIM), jnp.float32),
            pltpu.VMEM((CB, 16), jnp.float32),
            pltpu.SemaphoreType.DMA,
            pltpu.SemaphoreType.DMA,
        ],
    )
    def k(u_hbm, v_hbm, uidx_hbm, vidx_hbm, out_hbm,
          uidx_v, vidx_v, rows_v, vrows_v, part_v, sem0, sem1):
        wid = lax.axis_index("s") * 2 + lax.axis_index("c")
        base = wid * ROWS_PER_W
        sems = (sem0, sem1)

        def stage(ci, bufi):
            r0 = base + ci * CB
            pltpu.sync_copy(uidx_hbm.at[pl.ds(r0, CB)], uidx_v.at[bufi])
            pltpu.sync_copy(vidx_hbm.at[pl.ds(r0, CB)], vidx_v.at[bufi])
            pltpu.async_copy(v_hbm.at[vidx_v.at[bufi]], vrows_v.at[bufi],
                             sems[bufi])
            for b in range(CB):
                pltpu.async_copy(u_hbm.at[uidx_v.at[bufi, b]],
                                 rows_v.at[bufi, b], sems[bufi])

        def process(ci, bufi):
            r0 = base + ci * CB
            pltpu.make_async_copy(v_hbm.at[vidx_v.at[bufi]],
                                  vrows_v.at[bufi], sems[bufi]).wait()
            for b in range(CB):
                pltpu.make_async_copy(u_hbm.at[uidx_v.at[bufi, b]],
                                      rows_v.at[bufi, b], sems[bufi]).wait()

            def row_body(b, carry):
                a0 = rows_v[bufi, b, 0, pl.ds(0, 16)]
                a1 = rows_v[bufi, b, 0, pl.ds(16, 16)]
                a2 = rows_v[bufi, b, 0, pl.ds(32, 16)]
                a3 = rows_v[bufi, b, 0, pl.ds(48, 16)]
                for c in range(1, CTX):
                    a0 = a0 + rows_v[bufi, b, c, pl.ds(0, 16)]
                    a1 = a1 + rows_v[bufi, b, c, pl.ds(16, 16)]
                    a2 = a2 + rows_v[bufi, b, c, pl.ds(32, 16)]
                    a3 = a3 + rows_v[bufi, b, c, pl.ds(48, 16)]
                p = (a0 * vrows_v[bufi, b, pl.ds(0, 16)]
                     + a1 * vrows_v[bufi, b, pl.ds(16, 16)]
                     + a2 * vrows_v[bufi, b, pl.ds(32, 16)]
                     + a3 * vrows_v[bufi, b, pl.ds(48, 16)])
                part_v[b, :] = p
                return carry

            lax.fori_loop(0, CB, row_body, 0)
            pltpu.sync_copy(part_v, out_hbm.at[pl.ds(r0, CB)])

        stage(0, 0)

        def body2(h, carry):
            ci = 2 * h
            stage(ci + 1, 1)
            process(ci, 0)

            @pl.when(ci + 2 < CHUNKS)
            def _():
                stage(ci + 2, 0)

            process(ci + 1, 1)
            return carry

        lax.fori_loop(0, CHUNKS // 2, body2, 0)

    return k(u_weight, v_weight, all_u, all_v)


def _tc_loss(partials):
    def body(p_ref, o_ref):
        x = p_ref[...]
        s = jnp.sum(x, axis=1, keepdims=True)
        row = lax.broadcasted_iota(jnp.int32, (2 * BATCH, 1), 0)
        z = jnp.where(row < BATCH, s, -s)
        l = jnp.minimum(z, 0.0) - jnp.log1p(jnp.exp(-jnp.abs(z)))
        o_ref[0, 0] = -jnp.sum(l)

    out = pl.pallas_call(
        body,
        out_shape=jax.ShapeDtypeStruct((1, 1), jnp.float32),
        out_specs=pl.BlockSpec(memory_space=pltpu.SMEM),
    )(partials)
    return out[0, 0]


def kernel(pos_u, pos_v, neg_u, neg_v, u_weight, v_weight):
    all_u = jnp.concatenate([pos_u, neg_u], axis=0)
    all_v = jnp.concatenate([pos_v, neg_v], axis=0)
    partials = _sc_partials(u_weight, v_weight, all_u, all_v)
    return _tc_loss(partials)
